# baseline (speedup 1.0000x reference)
import functools

import jax
import jax.numpy as jnp
from jax import lax
from jax.experimental import pallas as pl
from jax.experimental.pallas import tpu as pltpu
from jax.experimental.pallas import tpu_sc as plsc

_VOCAB = 1000000
_EMBED = 64
_CTX = 200
_BATCH = 4096
_SEQ = 200

_NC = 2
_NS = 16
_NW = _NC * _NS
_SPW = _BATCH // _NW
_GCH = 100
_NCH = _SEQ // _GCH


def _emb_kernel(tok_hbm, e_hbm, p_hbm, out_hbm, idx_v, rows_v, p_v, sem):
    wid = lax.axis_index("s") * _NC + lax.axis_index("c")
    base = wid * _SPW
    pltpu.sync_copy(p_hbm, p_v)

    def seq_body(s, carry):
        pltpu.sync_copy(tok_hbm.at[base + s], idx_v)
        for h in range(_NCH):
            pltpu.async_copy(
                e_hbm.at[idx_v.at[h]],
                rows_v.at[pl.ds(h * _GCH, _GCH)],
                sem,
            ).wait()

        def add_body(i, c):
            for j in range(_EMBED // 16):
                sl = pl.ds(j * 16, 16)
                rows_v[i, sl] = rows_v[i, sl] + p_v[i, sl]
            return c

        lax.fori_loop(0, _SEQ, add_body, 0)
        pltpu.sync_copy(rows_v, out_hbm.at[base + s])
        return carry

    lax.fori_loop(0, _SPW, seq_body, 0)


def kernel(token_batch, E, P):
    tok = token_batch.reshape(_BATCH, _NCH, _GCH).astype(jnp.int32)
    mesh = plsc.VectorSubcoreMesh(core_axis_name="c", subcore_axis_name="s")
    run = functools.partial(
        pl.kernel,
        mesh=mesh,
        compiler_params=pltpu.CompilerParams(use_tc_tiling_on_sc=False),
        out_type=jax.ShapeDtypeStruct((_BATCH, _SEQ, _EMBED), jnp.float32),
        scratch_types=[
            pltpu.VMEM((_NCH, _GCH), jnp.int32),
            pltpu.VMEM((_SEQ, _EMBED), jnp.float32),
            pltpu.VMEM((_CTX, _EMBED), jnp.float32),
            pltpu.SemaphoreType.DMA,
        ],
    )(_emb_kernel)
    return run(tok, E, P)


# speedup vs baseline: 1.2497x; 1.2497x over previous
import functools

import jax
import jax.numpy as jnp
from jax import lax
from jax.experimental import pallas as pl
from jax.experimental.pallas import tpu as pltpu
from jax.experimental.pallas import tpu_sc as plsc

_VOCAB = 1000000
_EMBED = 64
_CTX = 200
_BATCH = 4096
_SEQ = 200

_NC = 2
_NS = 16
_NW = _NC * _NS
_CS = 4
_ROWS = _CS * _SEQ
_GCH = 100
_NG = _ROWS // _GCH
_CPW = _BATCH // (_NW * _CS)
_NB = 2


def _emb_kernel(tok_hbm, e_hbm, p_hbm, out_hbm,
                idx_v, rows_v, p_v, sem_g0, sem_g1, sem_s0, sem_s1):
    sem_g = (sem_g0, sem_g1)
    sem_s = (sem_s0, sem_s1)
    wid = lax.axis_index("s") * _NC + lax.axis_index("c")
    gbase = wid * _CPW
    pltpu.sync_copy(p_hbm, p_v)

    def fire_gathers(k, b):
        ib = idx_v.at[b]
        rb = rows_v.at[b]
        pltpu.sync_copy(tok_hbm.at[gbase + k], ib)
        for c in range(_NG):
            pltpu.async_copy(
                e_hbm.at[ib.at[c]],
                rb.at[pl.ds(c * _GCH, _GCH)],
                sem_g[b],
            )

    def wait_gathers(b):
        pltpu.make_async_copy(
            e_hbm.at[pl.ds(0, _ROWS)], rows_v.at[b], sem_g[b]
        ).wait()

    def wait_store(b):
        pltpu.make_async_copy(
            rows_v.at[b], out_hbm.at[0], sem_s[b]
        ).wait()

    def add_pos(b):
        rb = rows_v.at[b]

        def body(i, c):
            for j in range(_EMBED // 16):
                sl = pl.ds(j * 16, 16)
                pj = p_v[i, sl]
                for s in range(_CS):
                    plsc.addupdate(rb.at[s * _SEQ + i, sl], pj)
            return c

        lax.fori_loop(0, _SEQ, body, 0)

    fire_gathers(0, 0)

    def outer(i, carry):
        for b in range(_NB):
            k = _NB * i + b
            bn = b ^ 1

            @pl.when(k >= 1)
            def _():
                wait_store(bn)

            @pl.when(k + 1 < _CPW)
            def _():
                fire_gathers(k + 1, bn)

            wait_gathers(b)
            add_pos(b)
            pltpu.async_copy(rows_v.at[b], out_hbm.at[gbase + k], sem_s[b])
        return carry

    lax.fori_loop(0, _CPW // _NB, outer, 0)
    wait_store((_CPW - 1) % _NB)


def kernel(token_batch, E, P):
    tok = token_batch.reshape(_BATCH // _CS, _NG, _GCH).astype(jnp.int32)
    mesh = plsc.VectorSubcoreMesh(core_axis_name="c", subcore_axis_name="s")
    run = functools.partial(
        pl.kernel,
        mesh=mesh,
        compiler_params=pltpu.CompilerParams(use_tc_tiling_on_sc=False),
        out_type=jax.ShapeDtypeStruct((_BATCH // _CS, _ROWS, _EMBED),
                                      jnp.float32),
        scratch_types=[
            pltpu.VMEM((_NB, _NG, _GCH), jnp.int32),
            pltpu.VMEM((_NB, _ROWS, _EMBED), jnp.float32),
            pltpu.VMEM((_CTX, _EMBED), jnp.float32),
            pltpu.SemaphoreType.DMA,
            pltpu.SemaphoreType.DMA,
            pltpu.SemaphoreType.DMA,
            pltpu.SemaphoreType.DMA,
        ],
    )(_emb_kernel)
    out = run(tok, E, P)
    return out.reshape(_BATCH, _SEQ, _EMBED)
